# baseline (speedup 1.0000x reference)
import functools

import numpy as np
import jax
from jax import lax
import jax.numpy as jnp
from jax.experimental import pallas as pl
from jax.experimental.pallas import tpu as pltpu
from jax.experimental.pallas import tpu_sc as plsc


@functools.lru_cache
def _masked_positions(S):
    with jax.ensure_compile_time_eval():
        idx_arr = jax.random.choice(
            jax.random.key(42), S, shape=(3,), replace=False)
        return tuple(sorted(int(v) for v in np.asarray(idx_arr)))


def _tc_copy_body(nchunks, chunk, nbuf, x_ref, o_ref, *scr):
    bufs = scr[:nbuf]
    sin = scr[nbuf: 2 * nbuf]
    sout = scr[2 * nbuf: 3 * nbuf]

    def start_in(c):
        cpy = pltpu.make_async_copy(
            x_ref.at[pl.ds(c * chunk, chunk), :], bufs[c % nbuf],
            sin[c % nbuf])
        cpy.start()
        return cpy

    def start_out(c):
        cpy = pltpu.make_async_copy(
            bufs[c % nbuf], o_ref.at[pl.ds(c * chunk, chunk), :],
            sout[c % nbuf])
        cpy.start()
        return cpy

    in_h = {0: start_in(0)}
    out_h = {}
    for c in range(nchunks):
        in_h.pop(c).wait()
        out_h[c] = start_out(c)
        nxt = c + 1
        if nxt < nchunks:
            if nxt - nbuf in out_h:
                out_h.pop(nxt - nbuf).wait()
            in_h[nxt] = start_in(nxt)
    for c in sorted(out_h):
        out_h.pop(c).wait()


def _sc_scatter_body(mask_starts, upr, nc, o_ref, mt_ref, mtbuf, msem):
    wid = lax.axis_index("s") * nc + lax.axis_index("c")
    for k, r0 in enumerate(mask_starts):
        @pl.when(wid == k)
        def _():
            pltpu.async_copy(mt_ref, mtbuf, msem).wait()
            pltpu.async_copy(mtbuf, o_ref.at[pl.ds(r0, upr), :], msem).wait()


def kernel(x, mask_token):
    B, S, E = x.shape
    idx = _masked_positions(S)

    upr = E // 128
    R = B * S * upr
    mask_starts = [(b * S + s) * upr for b in range(B) for s in idx]

    f32 = jnp.float32
    xf = x.reshape(R, 128)
    mtf = mask_token.reshape(upr, 128)
    oref = jax.empty_ref(jax.ShapeDtypeStruct((R, 128), f32))

    chunk = 4096
    nbuf = 4
    nchunks = R // chunk
    tc_copy = pl.kernel(
        functools.partial(_tc_copy_body, nchunks, chunk, nbuf),
        mesh=pltpu.create_tensorcore_mesh("t"),
        out_type=(),
        scratch_types=(
            [pltpu.VMEM((chunk, 128), f32) for _ in range(nbuf)]
            + [pltpu.SemaphoreType.DMA for _ in range(2 * nbuf)]
        ),
    )
    tc_copy(xf, oref)

    info = plsc.get_sparse_core_info()
    nc = info.num_cores
    sc_scatter = pl.kernel(
        functools.partial(_sc_scatter_body, mask_starts, upr, nc),
        mesh=plsc.VectorSubcoreMesh(core_axis_name="c", subcore_axis_name="s"),
        out_type=(),
        scratch_types=[
            pltpu.VMEM((upr, 128), f32),
            pltpu.SemaphoreType.DMA,
        ],
    )
    sc_scatter(oref, mtf)
    return jax.freeze(oref).reshape(B, S, E)


# speedup vs baseline: 1.0364x; 1.0364x over previous
import functools

import numpy as np
import jax
from jax import lax
import jax.numpy as jnp
from jax.experimental import pallas as pl
from jax.experimental.pallas import tpu as pltpu
from jax.experimental.pallas import tpu_sc as plsc


@functools.lru_cache
def _masked_positions(S):
    with jax.ensure_compile_time_eval():
        idx_arr = jax.random.choice(
            jax.random.key(42), S, shape=(3,), replace=False)
        return tuple(sorted(int(v) for v in np.asarray(idx_arr)))


def _tc_copy_body(nchunks, chunk, nbuf, x_ref, o_ref, *scr):
    bufs = scr[:nbuf]
    sin = scr[nbuf: 2 * nbuf]
    sout = scr[2 * nbuf: 3 * nbuf]

    def start_in(c):
        cpy = pltpu.make_async_copy(
            x_ref.at[pl.ds(c * chunk, chunk), :], bufs[c % nbuf],
            sin[c % nbuf])
        cpy.start()
        return cpy

    def start_out(c):
        cpy = pltpu.make_async_copy(
            bufs[c % nbuf], o_ref.at[pl.ds(c * chunk, chunk), :],
            sout[c % nbuf])
        cpy.start()
        return cpy

    in_h = {0: start_in(0)}
    out_h = {}
    for c in range(nchunks):
        in_h.pop(c).wait()
        out_h[c] = start_out(c)
        nxt = c + 1
        if nxt < nchunks:
            if nxt - nbuf in out_h:
                out_h.pop(nxt - nbuf).wait()
            in_h[nxt] = start_in(nxt)
    for c in sorted(out_h):
        out_h.pop(c).wait()


def _sc_scatter_body(mask_spans, E, nc, o_ref, mt_ref, mtbuf, msem):
    wid = lax.axis_index("s") * nc + lax.axis_index("c")
    for k, (row, col) in enumerate(mask_spans):
        @pl.when(wid == k)
        def _():
            pltpu.async_copy(mt_ref, mtbuf, msem).wait()
            pltpu.async_copy(
                mtbuf, o_ref.at[row, pl.ds(col, E)], msem).wait()


def kernel(x, mask_token):
    B, S, E = x.shape
    idx = _masked_positions(S)

    W = 128 * 128
    R = (B * S * E) // W
    spr = W // E
    mask_spans = [(((b * S + s) // spr), ((b * S + s) % spr) * E)
                  for b in range(B) for s in idx]

    f32 = jnp.float32
    xf = x.reshape(R, W)
    mtf = mask_token.reshape(E)
    oref = jax.empty_ref(jax.ShapeDtypeStruct((R, W), f32))

    chunk = 128
    nbuf = 4
    nchunks = R // chunk
    tc_copy = pl.kernel(
        functools.partial(_tc_copy_body, nchunks, chunk, nbuf),
        mesh=pltpu.create_tensorcore_mesh("t"),
        out_type=(),
        scratch_types=(
            [pltpu.VMEM((chunk, W), f32) for _ in range(nbuf)]
            + [pltpu.SemaphoreType.DMA for _ in range(2 * nbuf)]
        ),
    )
    tc_copy(xf, oref)

    info = plsc.get_sparse_core_info()
    nc = info.num_cores
    sc_scatter = pl.kernel(
        functools.partial(_sc_scatter_body, mask_spans, E, nc),
        mesh=plsc.VectorSubcoreMesh(core_axis_name="c", subcore_axis_name="s"),
        out_type=(),
        scratch_types=[
            pltpu.VMEM((E,), f32),
            pltpu.SemaphoreType.DMA,
        ],
    )
    sc_scatter(oref, mtf)
    return jax.freeze(oref).reshape(B, S, E)


# speedup vs baseline: 1.0843x; 1.0463x over previous
import functools

import numpy as np
import jax
from jax import lax
import jax.numpy as jnp
from jax.experimental import pallas as pl
from jax.experimental.pallas import tpu as pltpu
from jax.experimental.pallas import tpu_sc as plsc


@functools.lru_cache
def _masked_positions(S):
    with jax.ensure_compile_time_eval():
        idx_arr = jax.random.choice(
            jax.random.key(42), S, shape=(3,), replace=False)
        return tuple(sorted(int(v) for v in np.asarray(idx_arr)))


def _tc_copy_body(nblocks, blk, W, x_ref, o_ref):
    def inner(x_blk, o_blk):
        o_blk[...] = x_blk[...]

    pltpu.emit_pipeline(
        inner,
        grid=(nblocks,),
        in_specs=[pl.BlockSpec((blk, W), lambda i: (i, 0))],
        out_specs=[pl.BlockSpec((blk, W), lambda i: (i, 0))],
    )(x_ref, o_ref)


def _sc_scatter_body(mask_spans, E, nc, o_ref, mt_ref, mtbuf, msem):
    wid = lax.axis_index("s") * nc + lax.axis_index("c")
    for k, (row, col) in enumerate(mask_spans):
        @pl.when(wid == k)
        def _():
            pltpu.async_copy(mt_ref, mtbuf, msem).wait()
            pltpu.async_copy(
                mtbuf, o_ref.at[row, pl.ds(col, E)], msem).wait()


def kernel(x, mask_token):
    B, S, E = x.shape
    idx = _masked_positions(S)

    W = 128 * 128
    R = (B * S * E) // W
    spr = W // E
    mask_spans = [(((b * S + s) // spr), ((b * S + s) % spr) * E)
                  for b in range(B) for s in idx]

    f32 = jnp.float32
    xf = x.reshape(R, W)
    mtf = mask_token.reshape(E)
    oref = jax.empty_ref(jax.ShapeDtypeStruct((R, W), f32))

    blk = 64
    nblocks = R // blk
    tc_copy = pl.kernel(
        functools.partial(_tc_copy_body, nblocks, blk, W),
        mesh=pltpu.create_tensorcore_mesh("t"),
        out_type=(),
    )
    tc_copy(xf, oref)

    info = plsc.get_sparse_core_info()
    nc = info.num_cores
    sc_scatter = pl.kernel(
        functools.partial(_sc_scatter_body, mask_spans, E, nc),
        mesh=plsc.VectorSubcoreMesh(core_axis_name="c", subcore_axis_name="s"),
        out_type=(),
        scratch_types=[
            pltpu.VMEM((E,), f32),
            pltpu.SemaphoreType.DMA,
        ],
    )
    sc_scatter(oref, mtf)
    return jax.freeze(oref).reshape(B, S, E)


# speedup vs baseline: 4.3212x; 3.9852x over previous
import functools

import numpy as np
import jax
from jax import lax
import jax.numpy as jnp
from jax.experimental import pallas as pl
from jax.experimental.pallas import tpu as pltpu
from jax.experimental.pallas import tpu_sc as plsc


@functools.lru_cache
def _masked_positions(S):
    with jax.ensure_compile_time_eval():
        idx_arr = jax.random.choice(
            jax.random.key(42), S, shape=(3,), replace=False)
        return tuple(sorted(int(v) for v in np.asarray(idx_arr)))


def _tc_copy_body(nblocks, blk, W, x_ref, o_ref):
    def inner(x_blk, o_blk):
        o_blk[...] = x_blk[...]

    pltpu.emit_pipeline(
        inner,
        grid=(nblocks,),
        in_specs=[pl.BlockSpec((blk, W), lambda i: (i, 0))],
        out_specs=[pl.BlockSpec((blk, W), lambda i: (i, 0))],
    )(x_ref, o_ref)


def _sc_scatter_body(mask_rows, E, nc, o_ref, mt_ref, mtbuf, msem):
    wid = lax.axis_index("s") * nc + lax.axis_index("c")
    for k, row in enumerate(mask_rows):
        @pl.when(wid == k)
        def _():
            pltpu.async_copy(mt_ref, mtbuf, msem).wait()
            pltpu.async_copy(
                mtbuf, o_ref.at[pl.ds(row, 1), :], msem).wait()


def kernel(x, mask_token):
    B, S, E = x.shape
    idx = _masked_positions(S)

    R = B * S
    mask_rows = [b * S + s for b in range(B) for s in idx]

    f32 = jnp.float32
    xf = x.reshape(R, E)
    mtf = mask_token.reshape(1, E)
    oref = jax.empty_ref(jax.ShapeDtypeStruct((R, E), f32))

    blk = 1024
    nblocks = R // blk
    tc_copy = pl.kernel(
        functools.partial(_tc_copy_body, nblocks, blk, E),
        mesh=pltpu.create_tensorcore_mesh("t"),
        out_type=(),
    )
    tc_copy(xf, oref)

    info = plsc.get_sparse_core_info()
    nc = info.num_cores
    sc_scatter = pl.kernel(
        functools.partial(_sc_scatter_body, mask_rows, E, nc),
        mesh=plsc.VectorSubcoreMesh(core_axis_name="c", subcore_axis_name="s"),
        out_type=(),
        scratch_types=[
            pltpu.VMEM((1, E), f32),
            pltpu.SemaphoreType.DMA,
        ],
        compiler_params=pltpu.CompilerParams(use_tc_tiling_on_sc=True),
    )
    sc_scatter(oref, mtf)
    return jax.freeze(oref).reshape(B, S, E)


# speedup vs baseline: 4.3958x; 1.0173x over previous
import functools

import numpy as np
import jax
from jax import lax
import jax.numpy as jnp
from jax.experimental import pallas as pl
from jax.experimental.pallas import tpu as pltpu
from jax.experimental.pallas import tpu_sc as plsc


@functools.lru_cache
def _masked_positions(S):
    with jax.ensure_compile_time_eval():
        idx_arr = jax.random.choice(
            jax.random.key(42), S, shape=(3,), replace=False)
        return tuple(sorted(int(v) for v in np.asarray(idx_arr)))


def _tc_copy_body(nblocks, blk, W, x_ref, o_ref):
    def inner(x_blk, o_blk):
        o_blk[...] = x_blk[...]

    pltpu.emit_pipeline(
        inner,
        grid=(nblocks,),
        in_specs=[pl.BlockSpec((blk, W), lambda i: (i, 0))],
        out_specs=[pl.BlockSpec((blk, W), lambda i: (i, 0))],
    )(x_ref, o_ref)


def _sc_scatter_body(mask_rows, E, nc, o_ref, mt_ref, mtbuf, msem):
    wid = lax.axis_index("s") * nc + lax.axis_index("c")
    for k, row in enumerate(mask_rows):
        @pl.when(wid == k)
        def _():
            pltpu.async_copy(mt_ref, mtbuf, msem).wait()
            pltpu.async_copy(
                mtbuf, o_ref.at[pl.ds(row, 1), :], msem).wait()


def kernel(x, mask_token):
    B, S, E = x.shape
    idx = _masked_positions(S)

    R = B * S
    mask_rows = [b * S + s for b in range(B) for s in idx]

    f32 = jnp.float32
    xf = x.reshape(R, E)
    mtf = mask_token.reshape(1, E)
    oref = jax.empty_ref(jax.ShapeDtypeStruct((R, E), f32))

    blk = 2048
    nblocks = R // blk
    tc_copy = pl.kernel(
        functools.partial(_tc_copy_body, nblocks, blk, E),
        mesh=pltpu.create_tensorcore_mesh("t"),
        out_type=(),
    )
    tc_copy(xf, oref)

    info = plsc.get_sparse_core_info()
    nc = info.num_cores
    sc_scatter = pl.kernel(
        functools.partial(_sc_scatter_body, mask_rows, E, nc),
        mesh=plsc.VectorSubcoreMesh(core_axis_name="c", subcore_axis_name="s"),
        out_type=(),
        scratch_types=[
            pltpu.VMEM((1, E), f32),
            pltpu.SemaphoreType.DMA,
        ],
        compiler_params=pltpu.CompilerParams(use_tc_tiling_on_sc=True),
    )
    sc_scatter(oref, mtf)
    return jax.freeze(oref).reshape(B, S, E)
